# baseline (speedup 1.0000x reference)
import functools

import jax
import jax.numpy as jnp
from jax import lax
from jax.experimental import pallas as pl
from jax.experimental.pallas import tpu as pltpu
from jax.experimental.pallas import tpu_sc as plsc

N_NODES = 10000
N_EDGES = 320000
D = 128

NC = 2
NS = 16
NW = NC * NS
EPW = N_EDGES // NW
CHUNK = 80
NCHUNK = EPW // CHUNK
NIDX = 4
RPT = 624

BLK = 2000


def _dense_body(x_ref, ws_ref, wd_ref, asrc_ref, adst_ref,
                z_ref, ss_ref, sd_ref):
    x = x_ref[...]
    z = lax.dot_general(x, ws_ref[...], (((1,), (1,)), ((), ())),
                        preferred_element_type=jnp.float32)
    z_ref[...] = z
    ss_ref[...] = lax.dot_general(z, asrc_ref[...], (((1,), (1,)), ((), ())),
                                  preferred_element_type=jnp.float32)
    zd = lax.dot_general(x, wd_ref[...], (((1,), (1,)), ((), ())),
                         preferred_element_type=jnp.float32)
    sd_ref[...] = lax.dot_general(zd, adst_ref[...], (((1,), (1,)), ((), ())),
                                  preferred_element_type=jnp.float32)


def _finish_body(h0_ref, h1_ref, d0_ref, d1_ref, o_ref):
    d = d0_ref[...] + d1_ref[...]
    d = jnp.where(d > 0.0, d, 1.0)
    o_ref[...] = (h0_ref[0] + h1_ref[0]) / d


_sc_mesh = plsc.VectorSubcoreMesh(core_axis_name="c", subcore_axis_name="s")


@functools.partial(
    pl.kernel,
    out_type=(
        jax.ShapeDtypeStruct((NC, N_NODES, D), jnp.float32),
        jax.ShapeDtypeStruct((NC, N_NODES), jnp.float32),
    ),
    mesh=_sc_mesh,
    compiler_params=pltpu.CompilerParams(needs_layout_passes=False),
    scratch_types=[
        pltpu.VMEM_SHARED((N_NODES, D), jnp.float32),
        pltpu.VMEM_SHARED((N_NODES,), jnp.float32),
        [pltpu.VMEM((2, CHUNK), jnp.int32) for _ in range(NIDX)],
        [pltpu.VMEM((CHUNK, D), jnp.float32) for _ in range(NIDX)],
        [pltpu.VMEM((CHUNK,), jnp.float32) for _ in range(NIDX)],
        [pltpu.VMEM((CHUNK,), jnp.float32) for _ in range(NIDX)],
        [pltpu.VMEM((CHUNK,), jnp.float32) for _ in range(NIDX)],
        pltpu.VMEM((16, D), jnp.float32),
        pltpu.VMEM((640,), jnp.float32),
        [pltpu.SemaphoreType.DMA for _ in range(NIDX)],
        [pltpu.SemaphoreType.DMA for _ in range(NIDX)],
        [pltpu.SemaphoreType.DMA for _ in range(NIDX)],
        [pltpu.SemaphoreType.DMA for _ in range(NIDX)],
    ],
)
def _edge_kernel(z_hbm, ssrc_hbm, sdst_hbm, srcc_hbm, dstc_hbm, hp_hbm, dp_hbm,
                 h_acc, d_acc, idx, rows, ssc, sdc, wb,
                 zbuf, dzbuf, isem, gsem, ssem, dsem):
    cid = lax.axis_index("c")
    sid = lax.axis_index("s")
    wid = cid * NS + sid

    zero16 = jnp.zeros((16,), jnp.float32)

    def zfill_row(r, c):
        for j in range(D // 16):
            zbuf[r, pl.ds(j * 16, 16)] = zero16
        return c
    lax.fori_loop(0, 16, zfill_row, 0)

    def zfill_d(i, c):
        dzbuf[pl.ds(i * 16, 16)] = zero16
        return c
    lax.fori_loop(0, 640 // 16, zfill_d, 0)

    def zcopy(k, c):
        pltpu.sync_copy(zbuf, h_acc.at[pl.ds(sid * RPT + k * 16, 16)])
        return c
    lax.fori_loop(0, RPT // 16, zcopy, 0)

    @pl.when(sid == NS - 1)
    def _():
        pltpu.sync_copy(zbuf, h_acc.at[pl.ds(NS * RPT, 16)])

    @pl.when(sid < NS - 1)
    def _():
        pltpu.sync_copy(dzbuf, d_acc.at[pl.ds(sid * 640, 640)])

    @pl.when(sid == NS - 1)
    def _():
        pltpu.sync_copy(dzbuf.at[pl.ds(0, 400)],
                        d_acc.at[pl.ds((NS - 1) * 640, 400)])

    plsc.subcore_barrier()

    gbase = wid * NCHUNK

    def issue_idx(q, j):
        pltpu.async_copy(srcc_hbm.at[gbase + q, 0], idx[j].at[0], isem[j])
        pltpu.async_copy(dstc_hbm.at[gbase + q, 0], idx[j].at[1], isem[j])

    def wait_idx(j):
        pltpu.make_async_copy(srcc_hbm.at[gbase, 0], idx[j].at[0], isem[j]).wait()
        pltpu.make_async_copy(dstc_hbm.at[gbase, 0], idx[j].at[1], isem[j]).wait()

    def issue_gather(j):
        pltpu.async_copy(z_hbm.at[idx[j].at[0]], rows[j], gsem[j])
        pltpu.async_copy(ssrc_hbm.at[idx[j].at[0]], ssc[j], gsem[j])
        pltpu.async_copy(sdst_hbm.at[idx[j].at[1]], sdc[j], gsem[j])

    def wait_gather(j):
        pltpu.make_async_copy(z_hbm.at[idx[j].at[0]], rows[j], gsem[j]).wait()
        pltpu.make_async_copy(ssrc_hbm.at[idx[j].at[0]], ssc[j], gsem[j]).wait()
        pltpu.make_async_copy(sdst_hbm.at[idx[j].at[1]], sdc[j], gsem[j]).wait()

    def issue_scatter(j):
        pltpu.async_copy(rows[j], h_acc.at[idx[j].at[1]], ssem[j], add=True)
        pltpu.async_copy(wb[j], d_acc.at[idx[j].at[1]], dsem[j], add=True)

    def wait_scatter(j):
        pltpu.make_async_copy(rows[j], h_acc.at[idx[j].at[1]], ssem[j]).wait()
        pltpu.make_async_copy(wb[j], d_acc.at[idx[j].at[1]], dsem[j]).wait()

    def step(q, j):
        j1 = (j + 1) % NIDX
        j2 = (j + 2) % NIDX

        @pl.when(q + 2 < NCHUNK)
        def _():
            @pl.when(q >= 2)
            def _():
                wait_scatter(j2)
            issue_idx(q + 2, j2)

        @pl.when(q + 1 < NCHUNK)
        def _():
            wait_idx(j1)
            issue_gather(j1)

        wait_gather(j)

        for i in range(CHUNK // 16):
            e = ssc[j][pl.ds(i * 16, 16)] + sdc[j][pl.ds(i * 16, 16)]
            e = jnp.where(e >= 0.0, e, 0.01 * e)
            wb[j][pl.ds(i * 16, 16)] = jnp.exp(e)

        @plsc.parallel_loop(0, CHUNK, unroll=8)
        def _(r):
            wr = plsc.load_gather(wb[j], [jnp.full((16,), r, jnp.int32)])
            for t in range(D // 16):
                rows[j][r, pl.ds(t * 16, 16)] = (
                    rows[j][r, pl.ds(t * 16, 16)] * wr)

        issue_scatter(j)

    issue_idx(0, 0)
    issue_idx(1, 1)
    wait_idx(0)
    issue_gather(0)

    def outer(m, c):
        for j in range(NIDX):
            step(m * NIDX + j, j)
        return c
    lax.fori_loop(0, (NCHUNK - 1) // NIDX, outer, 0)

    for q in range(((NCHUNK - 1) // NIDX) * NIDX, NCHUNK):
        step(q, q % NIDX)

    for j in range(NIDX):
        wait_scatter(j)

    plsc.subcore_barrier()

    pltpu.sync_copy(h_acc.at[pl.ds(sid * RPT, RPT)],
                    hp_hbm.at[cid, pl.ds(sid * RPT, RPT)])

    @pl.when(sid == NS - 1)
    def _():
        pltpu.sync_copy(h_acc.at[pl.ds(NS * RPT, 16)],
                        hp_hbm.at[cid, pl.ds(NS * RPT, 16)])

    @pl.when(sid == 0)
    def _():
        pltpu.sync_copy(d_acc, dp_hbm.at[cid])


def kernel(x, edge_index, W_src, W_dst, a_w):
    ei = edge_index.astype(jnp.int32)
    src_c = ei[0].reshape(NW * NCHUNK, 1, CHUNK)
    dst_c = ei[1].reshape(NW * NCHUNK, 1, CHUNK)
    a_src = a_w[:, :D]
    a_dst = a_w[:, D:]

    z_src, ss, sd = pl.pallas_call(
        _dense_body,
        grid=(N_NODES // BLK,),
        in_specs=[
            pl.BlockSpec((BLK, D), lambda i: (i, 0)),
            pl.BlockSpec((D, D), lambda i: (0, 0)),
            pl.BlockSpec((D, D), lambda i: (0, 0)),
            pl.BlockSpec((1, D), lambda i: (0, 0)),
            pl.BlockSpec((1, D), lambda i: (0, 0)),
        ],
        out_specs=[
            pl.BlockSpec((BLK, D), lambda i: (i, 0)),
            pl.BlockSpec((BLK, 1), lambda i: (i, 0)),
            pl.BlockSpec((BLK, 1), lambda i: (i, 0)),
        ],
        out_shape=[
            jax.ShapeDtypeStruct((N_NODES, D), jnp.float32),
            jax.ShapeDtypeStruct((N_NODES, 1), jnp.float32),
            jax.ShapeDtypeStruct((N_NODES, 1), jnp.float32),
        ],
    )(x, W_src, W_dst, a_src, a_dst)

    hp, dp = _edge_kernel(z_src, ss.reshape(N_NODES), sd.reshape(N_NODES),
                          src_c, dst_c)

    h = pl.pallas_call(
        _finish_body,
        grid=(N_NODES // BLK,),
        in_specs=[
            pl.BlockSpec((1, BLK, D), lambda i: (0, i, 0)),
            pl.BlockSpec((1, BLK, D), lambda i: (1, i, 0)),
            pl.BlockSpec((BLK, 1), lambda i: (i, 0)),
            pl.BlockSpec((BLK, 1), lambda i: (i, 0)),
        ],
        out_specs=pl.BlockSpec((BLK, D), lambda i: (i, 0)),
        out_shape=jax.ShapeDtypeStruct((N_NODES, D), jnp.float32),
    )(hp, hp, dp[0].reshape(N_NODES, 1), dp[1].reshape(N_NODES, 1))
    return h


# speedup vs baseline: 1.0583x; 1.0583x over previous
import functools

import jax
import jax.numpy as jnp
from jax import lax
from jax.experimental import pallas as pl
from jax.experimental.pallas import tpu as pltpu
from jax.experimental.pallas import tpu_sc as plsc

N_NODES = 10000
N_EDGES = 320000
D = 128

NC = 2
NS = 16
NW = NC * NS
EPW = N_EDGES // NW
CHUNK = 80
NCHUNK = EPW // CHUNK
NIDX = 4
NIDX8 = 8
RPT = 624

BLK = 2000


def _dense_body(x_ref, ws_ref, wd_ref, asrc_ref, adst_ref,
                z_ref, ss_ref, sd_ref):
    x = x_ref[...]
    z = lax.dot_general(x, ws_ref[...], (((1,), (1,)), ((), ())),
                        preferred_element_type=jnp.float32)
    z_ref[...] = z
    ss_ref[...] = lax.dot_general(z, asrc_ref[...], (((1,), (1,)), ((), ())),
                                  preferred_element_type=jnp.float32)
    zd = lax.dot_general(x, wd_ref[...], (((1,), (1,)), ((), ())),
                         preferred_element_type=jnp.float32)
    sd_ref[...] = lax.dot_general(zd, adst_ref[...], (((1,), (1,)), ((), ())),
                                  preferred_element_type=jnp.float32)


def _finish_body(h0_ref, h1_ref, d0_ref, d1_ref, o_ref):
    d = d0_ref[...] + d1_ref[...]
    d = jnp.where(d > 0.0, d, 1.0)
    o_ref[...] = (h0_ref[0] + h1_ref[0]) / d


_sc_mesh = plsc.VectorSubcoreMesh(core_axis_name="c", subcore_axis_name="s")


@functools.partial(
    pl.kernel,
    out_type=(
        jax.ShapeDtypeStruct((NC, N_NODES, D), jnp.float32),
        jax.ShapeDtypeStruct((NC, N_NODES), jnp.float32),
    ),
    mesh=_sc_mesh,
    compiler_params=pltpu.CompilerParams(needs_layout_passes=False),
    scratch_types=[
        pltpu.VMEM_SHARED((N_NODES, D), jnp.float32),
        pltpu.VMEM_SHARED((N_NODES,), jnp.float32),
        [pltpu.VMEM((2, CHUNK), jnp.int32) for _ in range(NIDX8)],
        [pltpu.VMEM((CHUNK, D), jnp.float32) for _ in range(NIDX)],
        [pltpu.VMEM((CHUNK,), jnp.float32) for _ in range(NIDX)],
        [pltpu.VMEM((CHUNK,), jnp.float32) for _ in range(NIDX)],
        [pltpu.VMEM((CHUNK,), jnp.float32) for _ in range(NIDX)],
        pltpu.VMEM((16, D), jnp.float32),
        pltpu.VMEM((640,), jnp.float32),
        [pltpu.SemaphoreType.DMA for _ in range(NIDX8)],
        [pltpu.SemaphoreType.DMA for _ in range(NIDX)],
        [pltpu.SemaphoreType.DMA for _ in range(NIDX)],
        [pltpu.SemaphoreType.DMA for _ in range(NIDX)],
    ],
)
def _edge_kernel(z_hbm, ssrc_hbm, sdst_hbm, srcc_hbm, dstc_hbm, hp_hbm, dp_hbm,
                 h_acc, d_acc, idx, rows, ssc, sdc, wb,
                 zbuf, dzbuf, isem, gsem, ssem, dsem):
    cid = lax.axis_index("c")
    sid = lax.axis_index("s")
    wid = cid * NS + sid

    zero16 = jnp.zeros((16,), jnp.float32)

    def zfill_row(r, c):
        for j in range(D // 16):
            zbuf[r, pl.ds(j * 16, 16)] = zero16
        return c
    lax.fori_loop(0, 16, zfill_row, 0)

    def zfill_d(i, c):
        dzbuf[pl.ds(i * 16, 16)] = zero16
        return c
    lax.fori_loop(0, 640 // 16, zfill_d, 0)

    def zcopy(k, c):
        pltpu.sync_copy(zbuf, h_acc.at[pl.ds(sid * RPT + k * 16, 16)])
        return c
    lax.fori_loop(0, RPT // 16, zcopy, 0)

    @pl.when(sid == NS - 1)
    def _():
        pltpu.sync_copy(zbuf, h_acc.at[pl.ds(NS * RPT, 16)])

    @pl.when(sid < NS - 1)
    def _():
        pltpu.sync_copy(dzbuf, d_acc.at[pl.ds(sid * 640, 640)])

    @pl.when(sid == NS - 1)
    def _():
        pltpu.sync_copy(dzbuf.at[pl.ds(0, 400)],
                        d_acc.at[pl.ds((NS - 1) * 640, 400)])

    plsc.subcore_barrier()

    gbase = wid * NCHUNK

    def issue_idx(q, j):
        pltpu.async_copy(srcc_hbm.at[gbase + q, 0], idx[j].at[0], isem[j])
        pltpu.async_copy(dstc_hbm.at[gbase + q, 0], idx[j].at[1], isem[j])

    def wait_idx(j):
        pltpu.make_async_copy(srcc_hbm.at[gbase, 0], idx[j].at[0], isem[j]).wait()
        pltpu.make_async_copy(dstc_hbm.at[gbase, 0], idx[j].at[1], isem[j]).wait()

    def issue_gather(ji, b):
        pltpu.async_copy(z_hbm.at[idx[ji].at[0]], rows[b], gsem[b])
        pltpu.async_copy(ssrc_hbm.at[idx[ji].at[0]], ssc[b], gsem[b])
        pltpu.async_copy(sdst_hbm.at[idx[ji].at[1]], sdc[b], gsem[b])

    def wait_gather(ji, b):
        pltpu.make_async_copy(z_hbm.at[idx[ji].at[0]], rows[b], gsem[b]).wait()
        pltpu.make_async_copy(ssrc_hbm.at[idx[ji].at[0]], ssc[b], gsem[b]).wait()
        pltpu.make_async_copy(sdst_hbm.at[idx[ji].at[1]], sdc[b], gsem[b]).wait()

    def issue_scatter(ji, b):
        pltpu.async_copy(rows[b], h_acc.at[idx[ji].at[1]], ssem[b], add=True)
        pltpu.async_copy(wb[b], d_acc.at[idx[ji].at[1]], dsem[b], add=True)

    def wait_scatter(ji, b):
        pltpu.make_async_copy(rows[b], h_acc.at[idx[ji].at[1]], ssem[b]).wait()
        pltpu.make_async_copy(wb[b], d_acc.at[idx[ji].at[1]], dsem[b]).wait()

    def step(q, j8):
        j4 = j8 % NIDX
        g2 = (j8 + 2) % NIDX
        i2 = (j8 + 2) % NIDX8
        i3 = (j8 + 3) % NIDX8

        @pl.when(q + 2 < NCHUNK)
        def _():
            @pl.when(q >= 2)
            def _():
                wait_scatter((j8 + 6) % NIDX8, g2)

            @pl.when(q + 3 < NCHUNK)
            def _():
                issue_idx(q + 3, i3)

            wait_idx(i2)
            issue_gather(i2, g2)

        wait_gather(j8, j4)

        for i in range(CHUNK // 16):
            e = ssc[j4][pl.ds(i * 16, 16)] + sdc[j4][pl.ds(i * 16, 16)]
            e = jnp.where(e >= 0.0, e, 0.01 * e)
            wb[j4][pl.ds(i * 16, 16)] = jnp.exp(e)

        @plsc.parallel_loop(0, CHUNK, unroll=4)
        def _(r):
            wr = plsc.load_gather(wb[j4], [jnp.full((16,), r, jnp.int32)])
            for t in range(D // 16):
                rows[j4][r, pl.ds(t * 16, 16)] = (
                    rows[j4][r, pl.ds(t * 16, 16)] * wr)

        issue_scatter(j8, j4)

    issue_idx(0, 0)
    issue_idx(1, 1)
    issue_idx(2, 2)
    wait_idx(0)
    issue_gather(0, 0)
    wait_idx(1)
    issue_gather(1, 1)

    def outer(m, c):
        for j8 in range(NIDX8):
            step(m * NIDX8 + j8, j8)
        return c
    lax.fori_loop(0, NCHUNK // NIDX8, outer, 0)

    for q in range((NCHUNK // NIDX8) * NIDX8, NCHUNK):
        step(jnp.int32(q), q % NIDX8)

    for q in range(NCHUNK - 4, NCHUNK):
        wait_scatter(q % NIDX8, q % NIDX)

    plsc.subcore_barrier()

    pltpu.sync_copy(h_acc.at[pl.ds(sid * RPT, RPT)],
                    hp_hbm.at[cid, pl.ds(sid * RPT, RPT)])

    @pl.when(sid == NS - 1)
    def _():
        pltpu.sync_copy(h_acc.at[pl.ds(NS * RPT, 16)],
                        hp_hbm.at[cid, pl.ds(NS * RPT, 16)])

    @pl.when(sid == 0)
    def _():
        pltpu.sync_copy(d_acc, dp_hbm.at[cid])


def kernel(x, edge_index, W_src, W_dst, a_w):
    ei = edge_index.astype(jnp.int32)
    src_c = ei[0].reshape(NW * NCHUNK, 1, CHUNK)
    dst_c = ei[1].reshape(NW * NCHUNK, 1, CHUNK)
    a_src = a_w[:, :D]
    a_dst = a_w[:, D:]

    z_src, ss, sd = pl.pallas_call(
        _dense_body,
        grid=(N_NODES // BLK,),
        in_specs=[
            pl.BlockSpec((BLK, D), lambda i: (i, 0)),
            pl.BlockSpec((D, D), lambda i: (0, 0)),
            pl.BlockSpec((D, D), lambda i: (0, 0)),
            pl.BlockSpec((1, D), lambda i: (0, 0)),
            pl.BlockSpec((1, D), lambda i: (0, 0)),
        ],
        out_specs=[
            pl.BlockSpec((BLK, D), lambda i: (i, 0)),
            pl.BlockSpec((BLK, 1), lambda i: (i, 0)),
            pl.BlockSpec((BLK, 1), lambda i: (i, 0)),
        ],
        out_shape=[
            jax.ShapeDtypeStruct((N_NODES, D), jnp.float32),
            jax.ShapeDtypeStruct((N_NODES, 1), jnp.float32),
            jax.ShapeDtypeStruct((N_NODES, 1), jnp.float32),
        ],
    )(x, W_src, W_dst, a_src, a_dst)

    hp, dp = _edge_kernel(z_src, ss.reshape(N_NODES), sd.reshape(N_NODES),
                          src_c, dst_c)

    h = pl.pallas_call(
        _finish_body,
        grid=(N_NODES // BLK,),
        in_specs=[
            pl.BlockSpec((1, BLK, D), lambda i: (0, i, 0)),
            pl.BlockSpec((1, BLK, D), lambda i: (1, i, 0)),
            pl.BlockSpec((BLK, 1), lambda i: (i, 0)),
            pl.BlockSpec((BLK, 1), lambda i: (i, 0)),
        ],
        out_specs=pl.BlockSpec((BLK, D), lambda i: (i, 0)),
        out_shape=jax.ShapeDtypeStruct((N_NODES, D), jnp.float32),
    )(hp, hp, dp[0].reshape(N_NODES, 1), dp[1].reshape(N_NODES, 1))
    return h
